# Initial kernel scaffold; baseline (speedup 1.0000x reference)
import jax
import jax.numpy as jnp
from jax.experimental import pallas as pl


def kernel(q, k, v):
    raise NotImplementedError("write your pallas kernel here")


# speedup vs baseline: 31.6156x; 31.6156x over previous
import functools
import math

import jax
import jax.numpy as jnp
from jax.experimental import pallas as pl
from jax.experimental.pallas import tpu as pltpu

_ALPHA = 0.2
_WINDOW = 128
_TOPK = 32
_LENS = 0.3

_BR = 256
_BN = 512


def _nth_largest(x, n):
    xm = x
    for _ in range(n - 1):
        m = jnp.max(xm, axis=-1, keepdims=True)
        xm = jnp.where(xm >= m, -jnp.inf, xm)
    return jnp.max(xm, axis=-1, keepdims=True)


def _stage1_kernel(q_ref, k_ref, v_ref, ylocal_ref, s_ref, *, topk, window,
                   alpha, br):
    i = pl.program_id(1)
    q = q_ref[0]
    k = k_ref[0]
    v = v_ref[0]
    n = k.shape[0]
    scale = 1.0 / math.sqrt(q.shape[-1])
    sim = jax.lax.dot_general(q, k, (((1,), (1,)), ((), ())),
                              preferred_element_type=jnp.float32) * scale
    rows = i * br + jax.lax.broadcasted_iota(jnp.int32, (br, n), 0)
    cols = jax.lax.broadcasted_iota(jnp.int32, (br, n), 1)
    in_band = jnp.abs(rows - cols) <= window
    a = jnp.where(in_band, sim, -jnp.inf)
    m = jnp.max(a, axis=-1, keepdims=True)
    p = jnp.exp(a - m)
    p = p / jnp.sum(p, axis=-1, keepdims=True)
    ylocal_ref[0] = jnp.dot(p, v, preferred_element_type=jnp.float32)
    x = jnp.where(in_band, -jnp.inf, sim)
    t32 = _nth_largest(x, topk)
    w = jnp.where((x >= t32) & (x > 0.0), x, 0.0)
    denom = jnp.sum(w, axis=-1, keepdims=True) + 1e-9
    s_ref[0] = (1.0 - alpha) * w / denom


def _mm_kernel(s_ref, t_ref, sij_ref, o_ref, acc_ref, *, bn, nk, first):
    i = pl.program_id(1)
    j = pl.program_id(2)
    kk = pl.program_id(3)
    @pl.when(kk == 0)
    def _():
        acc_ref[...] = jnp.zeros_like(acc_ref)
    acc_ref[...] += jnp.dot(s_ref[0], t_ref[0],
                            preferred_element_type=jnp.float32)
    @pl.when(kk == nk - 1)
    def _():
        rows = i * bn + jax.lax.broadcasted_iota(jnp.int32, (bn, bn), 0)
        cols = j * bn + jax.lax.broadcasted_iota(jnp.int32, (bn, bn), 1)
        eye = (rows == cols).astype(jnp.float32)
        out = acc_ref[...] + eye
        if first:
            out = out + sij_ref[0]
        o_ref[0] = out


def _stage4_kernel(u_ref, v_ref, ylocal_ref, o_ref, *, topk, coef):
    u = u_ref[0]
    t32 = _nth_largest(u, topk)
    w = jnp.where(u >= t32, u, 0.0)
    yr = jnp.dot(w, v_ref[0], preferred_element_type=jnp.float32)
    o_ref[0] = ylocal_ref[0] + coef * yr


def _matmul_chain_step(S, T, first, bn):
    Z, N, _ = S.shape
    nk = N // bn
    grid = (Z, N // bn, N // bn, nk)
    return pl.pallas_call(
        functools.partial(_mm_kernel, bn=bn, nk=nk, first=first),
        grid=grid,
        in_specs=[
            pl.BlockSpec((1, bn, bn), lambda z, i, j, kk: (z, i, kk)),
            pl.BlockSpec((1, bn, bn), lambda z, i, j, kk: (z, kk, j)),
            pl.BlockSpec((1, bn, bn), lambda z, i, j, kk: (z, i, j)),
        ],
        out_specs=pl.BlockSpec((1, bn, bn), lambda z, i, j, kk: (z, i, j)),
        out_shape=jax.ShapeDtypeStruct((Z, N, N), jnp.float32),
        scratch_shapes=[pltpu.VMEM((bn, bn), jnp.float32)],
        compiler_params=pltpu.CompilerParams(
            dimension_semantics=("parallel", "parallel", "parallel",
                                 "arbitrary")),
    )(S, T, S)


def kernel(q, k, v):
    B, H, N, D = q.shape
    Z = B * H
    br = min(_BR, N)
    bn = min(_BN, N)
    qf = q.reshape(Z, N, D)
    kf = k.reshape(Z, N, D)
    vf = v.reshape(Z, N, D)

    ylocal, S = pl.pallas_call(
        functools.partial(_stage1_kernel, topk=_TOPK, window=_WINDOW,
                          alpha=_ALPHA, br=br),
        grid=(Z, N // br),
        in_specs=[
            pl.BlockSpec((1, br, D), lambda z, i: (z, i, 0)),
            pl.BlockSpec((1, N, D), lambda z, i: (z, 0, 0)),
            pl.BlockSpec((1, N, D), lambda z, i: (z, 0, 0)),
        ],
        out_specs=[
            pl.BlockSpec((1, br, D), lambda z, i: (z, i, 0)),
            pl.BlockSpec((1, br, N), lambda z, i: (z, i, 0)),
        ],
        out_shape=[
            jax.ShapeDtypeStruct((Z, N, D), jnp.float32),
            jax.ShapeDtypeStruct((Z, N, N), jnp.float32),
        ],
        compiler_params=pltpu.CompilerParams(
            dimension_semantics=("parallel", "parallel")),
    )(qf, kf, vf)

    T = _matmul_chain_step(S, S, True, bn)
    T = _matmul_chain_step(S, T, False, bn)
    U = _matmul_chain_step(S, T, False, bn)

    out = pl.pallas_call(
        functools.partial(_stage4_kernel, topk=_TOPK, coef=_LENS * _ALPHA),
        grid=(Z, N // br),
        in_specs=[
            pl.BlockSpec((1, br, N), lambda z, i: (z, i, 0)),
            pl.BlockSpec((1, N, D), lambda z, i: (z, 0, 0)),
            pl.BlockSpec((1, br, D), lambda z, i: (z, i, 0)),
        ],
        out_specs=pl.BlockSpec((1, br, D), lambda z, i: (z, i, 0)),
        out_shape=jax.ShapeDtypeStruct((Z, N, D), jnp.float32),
        compiler_params=pltpu.CompilerParams(
            dimension_semantics=("parallel", "parallel")),
    )(U, vf, ylocal)

    return out.reshape(B, H, N, D)


# speedup vs baseline: 31.6673x; 1.0016x over previous
import functools
import math

import jax
import jax.numpy as jnp
from jax.experimental import pallas as pl
from jax.experimental.pallas import tpu as pltpu

_ALPHA = 0.2
_WINDOW = 128
_TOPK = 32
_LENS = 0.3

_BR = 256
_BN = 512


def _nth_largest(x, n):
    xm = x
    for _ in range(n - 1):
        m = jnp.max(xm, axis=-1, keepdims=True)
        xm = jnp.where(xm >= m, -jnp.inf, xm)
    return jnp.max(xm, axis=-1, keepdims=True)


def _stage1_kernel(q_ref, k_ref, v_ref, ylocal_ref, s_ref, *, topk, window,
                   alpha, br):
    i = pl.program_id(1)
    q = q_ref[0]
    k = k_ref[0]
    v = v_ref[0]
    n = k.shape[0]
    scale = 1.0 / math.sqrt(q.shape[-1])
    sim = jax.lax.dot_general(q, k, (((1,), (1,)), ((), ())),
                              preferred_element_type=jnp.float32) * scale
    rows = i * br + jax.lax.broadcasted_iota(jnp.int32, (br, n), 0)
    cols = jax.lax.broadcasted_iota(jnp.int32, (br, n), 1)
    in_band = jnp.abs(rows - cols) <= window
    a = jnp.where(in_band, sim, -jnp.inf)
    m = jnp.max(a, axis=-1, keepdims=True)
    p = jnp.exp(a - m)
    p = p / jnp.sum(p, axis=-1, keepdims=True)
    ylocal_ref[0] = jnp.dot(p, v, preferred_element_type=jnp.float32)
    x = jnp.where(in_band, -jnp.inf, sim)
    t32 = _nth_largest(x, topk)
    w = jnp.where((x >= t32) & (x > 0.0), x, 0.0)
    denom = jnp.sum(w, axis=-1, keepdims=True) + 1e-9
    s_ref[0] = (1.0 - alpha) * w / denom


def _mm_kernel(s_ref, t_ref, sij_ref, o_ref, acc_ref, *, bn, nk, first):
    i = pl.program_id(1)
    j = pl.program_id(2)
    kk = pl.program_id(3)
    @pl.when(kk == 0)
    def _():
        acc_ref[...] = jnp.zeros_like(acc_ref)
    acc_ref[...] += jnp.dot(s_ref[0].astype(jnp.bfloat16),
                            t_ref[0].astype(jnp.bfloat16),
                            preferred_element_type=jnp.float32)
    @pl.when(kk == nk - 1)
    def _():
        rows = i * bn + jax.lax.broadcasted_iota(jnp.int32, (bn, bn), 0)
        cols = j * bn + jax.lax.broadcasted_iota(jnp.int32, (bn, bn), 1)
        eye = (rows == cols).astype(jnp.float32)
        out = acc_ref[...] + eye
        if first:
            out = out + sij_ref[0]
        o_ref[0] = out


def _stage4_kernel(u_ref, v_ref, ylocal_ref, o_ref, *, topk, coef):
    u = u_ref[0]
    t32 = _nth_largest(u, topk)
    w = jnp.where(u >= t32, u, 0.0)
    yr = jnp.dot(w, v_ref[0], preferred_element_type=jnp.float32)
    o_ref[0] = ylocal_ref[0] + coef * yr


def _matmul_chain_step(S, T, first, bn):
    Z, N, _ = S.shape
    nk = N // bn
    grid = (Z, N // bn, N // bn, nk)
    return pl.pallas_call(
        functools.partial(_mm_kernel, bn=bn, nk=nk, first=first),
        grid=grid,
        in_specs=[
            pl.BlockSpec((1, bn, bn), lambda z, i, j, kk: (z, i, kk)),
            pl.BlockSpec((1, bn, bn), lambda z, i, j, kk: (z, kk, j)),
            pl.BlockSpec((1, bn, bn), lambda z, i, j, kk: (z, i, j)),
        ],
        out_specs=pl.BlockSpec((1, bn, bn), lambda z, i, j, kk: (z, i, j)),
        out_shape=jax.ShapeDtypeStruct((Z, N, N), jnp.float32),
        scratch_shapes=[pltpu.VMEM((bn, bn), jnp.float32)],
        compiler_params=pltpu.CompilerParams(
            dimension_semantics=("parallel", "parallel", "parallel",
                                 "arbitrary")),
    )(S, T, S)


def kernel(q, k, v):
    B, H, N, D = q.shape
    Z = B * H
    br = min(_BR, N)
    bn = min(_BN, N)
    qf = q.reshape(Z, N, D)
    kf = k.reshape(Z, N, D)
    vf = v.reshape(Z, N, D)

    ylocal, S = pl.pallas_call(
        functools.partial(_stage1_kernel, topk=_TOPK, window=_WINDOW,
                          alpha=_ALPHA, br=br),
        grid=(Z, N // br),
        in_specs=[
            pl.BlockSpec((1, br, D), lambda z, i: (z, i, 0)),
            pl.BlockSpec((1, N, D), lambda z, i: (z, 0, 0)),
            pl.BlockSpec((1, N, D), lambda z, i: (z, 0, 0)),
        ],
        out_specs=[
            pl.BlockSpec((1, br, D), lambda z, i: (z, i, 0)),
            pl.BlockSpec((1, br, N), lambda z, i: (z, i, 0)),
        ],
        out_shape=[
            jax.ShapeDtypeStruct((Z, N, D), jnp.float32),
            jax.ShapeDtypeStruct((Z, N, N), jnp.float32),
        ],
        compiler_params=pltpu.CompilerParams(
            dimension_semantics=("parallel", "parallel")),
    )(qf, kf, vf)

    T = _matmul_chain_step(S, S, True, bn)
    T = _matmul_chain_step(S, T, False, bn)
    U = _matmul_chain_step(S, T, False, bn)

    out = pl.pallas_call(
        functools.partial(_stage4_kernel, topk=_TOPK, coef=_LENS * _ALPHA),
        grid=(Z, N // br),
        in_specs=[
            pl.BlockSpec((1, br, N), lambda z, i: (z, i, 0)),
            pl.BlockSpec((1, N, D), lambda z, i: (z, 0, 0)),
            pl.BlockSpec((1, br, D), lambda z, i: (z, i, 0)),
        ],
        out_specs=pl.BlockSpec((1, br, D), lambda z, i: (z, i, 0)),
        out_shape=jax.ShapeDtypeStruct((Z, N, D), jnp.float32),
        compiler_params=pltpu.CompilerParams(
            dimension_semantics=("parallel", "parallel")),
    )(U, vf, ylocal)

    return out.reshape(B, H, N, D)


# speedup vs baseline: 55.1925x; 1.7429x over previous
import functools
import math

import jax
import jax.numpy as jnp
from jax.experimental import pallas as pl
from jax.experimental.pallas import tpu as pltpu

_ALPHA = 0.2
_WINDOW = 128
_TOPK = 32
_LENS = 0.3

_BR = 256
_BRM = 512
_BK = 512


def _nth_largest(x, n):
    xm = x
    for _ in range(n - 1):
        m = jnp.max(xm, axis=-1, keepdims=True)
        xm = jnp.where(xm >= m, -jnp.inf, xm)
    return jnp.max(xm, axis=-1, keepdims=True)


def _stage1_kernel(q_ref, k_ref, v_ref, ylocal_ref, s_ref, t1_ref, *, topk,
                   window, alpha, br):
    i = pl.program_id(1)
    q = q_ref[0]
    k = k_ref[0]
    n = k.shape[0]
    scale = 1.0 / math.sqrt(q.shape[-1])
    sim = jax.lax.dot_general(q, k, (((1,), (1,)), ((), ())),
                              preferred_element_type=jnp.float32) * scale
    rows = i * br + jax.lax.broadcasted_iota(jnp.int32, (br, n), 0)
    cols = jax.lax.broadcasted_iota(jnp.int32, (br, n), 1)
    in_band = jnp.abs(rows - cols) <= window
    slab = br + 2 * window if br + 2 * window <= n else n
    start = jnp.clip(i * br - (slab - br) // 2, 0, n - slab)
    ks = k_ref[0, pl.ds(start, slab), :]
    vs = v_ref[0, pl.ds(start, slab), :]
    sl = jax.lax.dot_general(q, ks, (((1,), (1,)), ((), ())),
                             preferred_element_type=jnp.float32) * scale
    srows = i * br + jax.lax.broadcasted_iota(jnp.int32, (br, slab), 0)
    scols = start + jax.lax.broadcasted_iota(jnp.int32, (br, slab), 1)
    sband = jnp.abs(srows - scols) <= window
    a = jnp.where(sband, sl, -jnp.inf)
    m = jnp.max(a, axis=-1, keepdims=True)
    p = jnp.exp(a - m)
    p = p * (1.0 / jnp.sum(p, axis=-1, keepdims=True))
    ylocal_ref[0] = jnp.dot(p, vs, preferred_element_type=jnp.float32)
    x = jnp.where(in_band, -jnp.inf, sim)
    t32 = _nth_largest(x, topk)
    w = jnp.where((x >= t32) & (x > 0.0), x, 0.0)
    denom = jnp.sum(w, axis=-1, keepdims=True) + 1e-9
    s = (1.0 - alpha) * w / denom
    s_ref[0] = s.astype(jnp.bfloat16)
    eye = (rows == cols).astype(jnp.float32)
    t1_ref[0] = (s + eye).astype(jnp.bfloat16)


def _mm_kernel(s_ref, t_ref, o_ref, acc_ref, *, nk, br):
    i = pl.program_id(1)
    kk = pl.program_id(2)
    @pl.when(kk == 0)
    def _():
        acc_ref[...] = jnp.zeros_like(acc_ref)
    acc_ref[...] += jnp.dot(s_ref[0], t_ref[0],
                            preferred_element_type=jnp.float32)
    @pl.when(kk == nk - 1)
    def _():
        n = acc_ref.shape[-1]
        rows = i * br + jax.lax.broadcasted_iota(jnp.int32, (br, n), 0)
        cols = jax.lax.broadcasted_iota(jnp.int32, (br, n), 1)
        eye = (rows == cols).astype(jnp.float32)
        o_ref[0] = (acc_ref[...] + eye).astype(jnp.bfloat16)


def _mm_final_kernel(s_ref, t_ref, v_ref, ylocal_ref, o_ref, acc_ref, *, nk,
                     br, topk, coef):
    i = pl.program_id(1)
    kk = pl.program_id(2)
    @pl.when(kk == 0)
    def _():
        acc_ref[...] = jnp.zeros_like(acc_ref)
    acc_ref[...] += jnp.dot(s_ref[0], t_ref[0],
                            preferred_element_type=jnp.float32)
    @pl.when(kk == nk - 1)
    def _():
        n = acc_ref.shape[-1]
        rows = i * br + jax.lax.broadcasted_iota(jnp.int32, (br, n), 0)
        cols = jax.lax.broadcasted_iota(jnp.int32, (br, n), 1)
        u = acc_ref[...] + (rows == cols).astype(jnp.float32)
        t32 = _nth_largest(u, topk)
        w = jnp.where(u >= t32, u, 0.0)
        yr = jnp.dot(w, v_ref[0], preferred_element_type=jnp.float32)
        o_ref[0] = ylocal_ref[0] + coef * yr


def _matmul_chain_step(S, T, brm, bk):
    Z, N, _ = S.shape
    nk = N // bk
    return pl.pallas_call(
        functools.partial(_mm_kernel, nk=nk, br=brm),
        grid=(Z, N // brm, nk),
        in_specs=[
            pl.BlockSpec((1, brm, bk), lambda z, i, kk: (z, i, kk)),
            pl.BlockSpec((1, bk, N), lambda z, i, kk: (z, kk, 0)),
        ],
        out_specs=pl.BlockSpec((1, brm, N), lambda z, i, kk: (z, i, 0)),
        out_shape=jax.ShapeDtypeStruct((Z, N, N), jnp.bfloat16),
        scratch_shapes=[pltpu.VMEM((brm, N), jnp.float32)],
        compiler_params=pltpu.CompilerParams(
            dimension_semantics=("parallel", "parallel", "arbitrary")),
    )(S, T)


def kernel(q, k, v):
    B, H, N, D = q.shape
    Z = B * H
    br = min(_BR, N)
    brm = min(_BRM, N)
    bk = min(_BK, N)
    qf = q.reshape(Z, N, D)
    kf = k.reshape(Z, N, D)
    vf = v.reshape(Z, N, D)

    ylocal, S, T1 = pl.pallas_call(
        functools.partial(_stage1_kernel, topk=_TOPK, window=_WINDOW,
                          alpha=_ALPHA, br=br),
        grid=(Z, N // br),
        in_specs=[
            pl.BlockSpec((1, br, D), lambda z, i: (z, i, 0)),
            pl.BlockSpec((1, N, D), lambda z, i: (z, 0, 0)),
            pl.BlockSpec((1, N, D), lambda z, i: (z, 0, 0)),
        ],
        out_specs=[
            pl.BlockSpec((1, br, D), lambda z, i: (z, i, 0)),
            pl.BlockSpec((1, br, N), lambda z, i: (z, i, 0)),
            pl.BlockSpec((1, br, N), lambda z, i: (z, i, 0)),
        ],
        out_shape=[
            jax.ShapeDtypeStruct((Z, N, D), jnp.float32),
            jax.ShapeDtypeStruct((Z, N, N), jnp.bfloat16),
            jax.ShapeDtypeStruct((Z, N, N), jnp.bfloat16),
        ],
        compiler_params=pltpu.CompilerParams(
            dimension_semantics=("parallel", "parallel")),
    )(qf, kf, vf)

    T = _matmul_chain_step(S, T1, brm, bk)
    T = _matmul_chain_step(S, T, brm, bk)

    nk = N // bk
    out = pl.pallas_call(
        functools.partial(_mm_final_kernel, nk=nk, br=brm, topk=_TOPK,
                          coef=_LENS * _ALPHA),
        grid=(Z, N // brm, nk),
        in_specs=[
            pl.BlockSpec((1, brm, bk), lambda z, i, kk: (z, i, kk)),
            pl.BlockSpec((1, bk, N), lambda z, i, kk: (z, kk, 0)),
            pl.BlockSpec((1, N, D), lambda z, i, kk: (z, 0, 0)),
            pl.BlockSpec((1, brm, D), lambda z, i, kk: (z, i, 0)),
        ],
        out_specs=pl.BlockSpec((1, brm, D), lambda z, i, kk: (z, i, 0)),
        out_shape=jax.ShapeDtypeStruct((Z, N, D), jnp.float32),
        scratch_shapes=[pltpu.VMEM((brm, N), jnp.float32)],
        compiler_params=pltpu.CompilerParams(
            dimension_semantics=("parallel", "parallel", "arbitrary")),
    )(S, T, vf, ylocal)

    return out.reshape(B, H, N, D)


# speedup vs baseline: 63.7062x; 1.1543x over previous
import functools
import math

import jax
import jax.numpy as jnp
from jax.experimental import pallas as pl
from jax.experimental.pallas import tpu as pltpu

_ALPHA = 0.2
_WINDOW = 128
_TOPK = 32
_LENS = 0.3

_BR = 256
_BRM = 512
_BK = 512


def _nth_largest(x, n):
    xm = x
    for _ in range(n - 1):
        m = jnp.max(xm, axis=-1, keepdims=True)
        xm = jnp.where(xm >= m, -jnp.inf, xm)
    return jnp.max(xm, axis=-1, keepdims=True)


def _stage1_kernel(q_ref, k_ref, v_ref, ylocal_ref, s_ref, *, topk,
                   window, alpha, br):
    i = pl.program_id(1)
    q = q_ref[0]
    k = k_ref[0]
    n = k.shape[0]
    scale = 1.0 / math.sqrt(q.shape[-1])
    sim = jax.lax.dot_general(q, k, (((1,), (1,)), ((), ())),
                              preferred_element_type=jnp.float32) * scale
    rows = i * br + jax.lax.broadcasted_iota(jnp.int32, (br, n), 0)
    cols = jax.lax.broadcasted_iota(jnp.int32, (br, n), 1)
    in_band = jnp.abs(rows - cols) <= window
    slab = br + 2 * window if br + 2 * window <= n else n
    start = jnp.clip(i * br - (slab - br) // 2, 0, n - slab)
    ks = k_ref[0, pl.ds(start, slab), :]
    vs = v_ref[0, pl.ds(start, slab), :]
    sl = jax.lax.dot_general(q, ks, (((1,), (1,)), ((), ())),
                             preferred_element_type=jnp.float32) * scale
    srows = i * br + jax.lax.broadcasted_iota(jnp.int32, (br, slab), 0)
    scols = start + jax.lax.broadcasted_iota(jnp.int32, (br, slab), 1)
    sband = jnp.abs(srows - scols) <= window
    a = jnp.where(sband, sl, -jnp.inf)
    m = jnp.max(a, axis=-1, keepdims=True)
    p = jnp.exp(a - m)
    p = p * (1.0 / jnp.sum(p, axis=-1, keepdims=True))
    ylocal_ref[0] = jnp.dot(p, vs, preferred_element_type=jnp.float32)
    x = jnp.where(in_band, -jnp.inf, sim)
    t32 = _nth_largest(x, topk)
    w = jnp.where((x >= t32) & (x > 0.0), x, 0.0)
    denom = jnp.sum(w, axis=-1, keepdims=True) + 1e-9
    s_ref[0] = ((1.0 - alpha) * w / denom).astype(jnp.bfloat16)


def _mm1_kernel(sl_ref, sr_ref, a_ref, b_ref, acc_ref, srow_ref, *, nk):
    i = pl.program_id(1)
    kk = pl.program_id(2)
    @pl.when(kk == 0)
    def _():
        acc_ref[...] = jnp.zeros_like(acc_ref)
    acc_ref[...] += jnp.dot(sl_ref[0], sr_ref[0],
                            preferred_element_type=jnp.float32)
    @pl.when(kk == i)
    def _():
        srow_ref[...] = sr_ref[0]
    @pl.when(kk == nk - 1)
    def _():
        acc = acc_ref[...]
        a_ref[0] = acc.astype(jnp.bfloat16)
        b_ref[0] = (acc + srow_ref[...].astype(jnp.float32)).astype(
            jnp.bfloat16)


def _mm2_final_kernel(a_ref, b_ref, v_ref, ylocal_ref, o_ref, acc_ref,
                      brow_ref, *, nk, br, topk, coef):
    i = pl.program_id(1)
    kk = pl.program_id(2)
    @pl.when(kk == 0)
    def _():
        acc_ref[...] = jnp.zeros_like(acc_ref)
    acc_ref[...] += jnp.dot(a_ref[0], b_ref[0],
                            preferred_element_type=jnp.float32)
    @pl.when(kk == i)
    def _():
        brow_ref[...] = b_ref[0]
    @pl.when(kk == nk - 1)
    def _():
        n = acc_ref.shape[-1]
        rows = i * br + jax.lax.broadcasted_iota(jnp.int32, (br, n), 0)
        cols = jax.lax.broadcasted_iota(jnp.int32, (br, n), 1)
        u = (acc_ref[...] + brow_ref[...].astype(jnp.float32)
             + (rows == cols).astype(jnp.float32))
        t32 = _nth_largest(u, topk)
        w = jnp.where(u >= t32, u, 0.0)
        yr = jnp.dot(w, v_ref[0], preferred_element_type=jnp.float32)
        o_ref[0] = ylocal_ref[0] + coef * yr


def kernel(q, k, v):
    B, H, N, D = q.shape
    Z = B * H
    br = min(_BR, N)
    brm = min(_BRM, N)
    bk = brm
    qf = q.reshape(Z, N, D)
    kf = k.reshape(Z, N, D)
    vf = v.reshape(Z, N, D)

    ylocal, S = pl.pallas_call(
        functools.partial(_stage1_kernel, topk=_TOPK, window=_WINDOW,
                          alpha=_ALPHA, br=br),
        grid=(Z, N // br),
        in_specs=[
            pl.BlockSpec((1, br, D), lambda z, i: (z, i, 0)),
            pl.BlockSpec((1, N, D), lambda z, i: (z, 0, 0)),
            pl.BlockSpec((1, N, D), lambda z, i: (z, 0, 0)),
        ],
        out_specs=[
            pl.BlockSpec((1, br, D), lambda z, i: (z, i, 0)),
            pl.BlockSpec((1, br, N), lambda z, i: (z, i, 0)),
        ],
        out_shape=[
            jax.ShapeDtypeStruct((Z, N, D), jnp.float32),
            jax.ShapeDtypeStruct((Z, N, N), jnp.bfloat16),
        ],
        compiler_params=pltpu.CompilerParams(
            dimension_semantics=("parallel", "parallel")),
    )(qf, kf, vf)

    nk = N // bk
    A, Bm = pl.pallas_call(
        functools.partial(_mm1_kernel, nk=nk),
        grid=(Z, N // brm, nk),
        in_specs=[
            pl.BlockSpec((1, brm, bk), lambda z, i, kk: (z, i, kk)),
            pl.BlockSpec((1, bk, N), lambda z, i, kk: (z, kk, 0)),
        ],
        out_specs=[
            pl.BlockSpec((1, brm, N), lambda z, i, kk: (z, i, 0)),
            pl.BlockSpec((1, brm, N), lambda z, i, kk: (z, i, 0)),
        ],
        out_shape=[
            jax.ShapeDtypeStruct((Z, N, N), jnp.bfloat16),
            jax.ShapeDtypeStruct((Z, N, N), jnp.bfloat16),
        ],
        scratch_shapes=[pltpu.VMEM((brm, N), jnp.float32),
                        pltpu.VMEM((bk, N), jnp.bfloat16)],
        compiler_params=pltpu.CompilerParams(
            dimension_semantics=("parallel", "parallel", "arbitrary")),
    )(S, S)

    out = pl.pallas_call(
        functools.partial(_mm2_final_kernel, nk=nk, br=brm, topk=_TOPK,
                          coef=_LENS * _ALPHA),
        grid=(Z, N // brm, nk),
        in_specs=[
            pl.BlockSpec((1, brm, bk), lambda z, i, kk: (z, i, kk)),
            pl.BlockSpec((1, bk, N), lambda z, i, kk: (z, kk, 0)),
            pl.BlockSpec((1, N, D), lambda z, i, kk: (z, 0, 0)),
            pl.BlockSpec((1, brm, D), lambda z, i, kk: (z, i, 0)),
        ],
        out_specs=pl.BlockSpec((1, brm, D), lambda z, i, kk: (z, i, 0)),
        out_shape=jax.ShapeDtypeStruct((Z, N, D), jnp.float32),
        scratch_shapes=[pltpu.VMEM((brm, N), jnp.float32),
                        pltpu.VMEM((bk, N), jnp.bfloat16)],
        compiler_params=pltpu.CompilerParams(
            dimension_semantics=("parallel", "parallel", "arbitrary")),
    )(A, Bm, vf, ylocal)

    return out.reshape(B, H, N, D)


# speedup vs baseline: 68.8019x; 1.0800x over previous
import functools
import math

import jax
import jax.numpy as jnp
from jax.experimental import pallas as pl
from jax.experimental.pallas import tpu as pltpu

_ALPHA = 0.2
_WINDOW = 128
_TOPK = 32
_LENS = 0.3

_BR = 512
_BRM = 1024
_BK = 1024


def _nth_largest(x, n):
    xm = x
    for _ in range(n - 1):
        m = jnp.max(xm, axis=-1, keepdims=True)
        xm = jnp.where(xm >= m, -jnp.inf, xm)
    return jnp.max(xm, axis=-1, keepdims=True)


def _stage1_kernel(q_ref, k_ref, v_ref, ylocal_ref, s_ref, *, topk,
                   window, alpha, br):
    i = pl.program_id(1)
    q = q_ref[0]
    k = k_ref[0]
    n = k.shape[0]
    scale = 1.0 / math.sqrt(q.shape[-1])
    sim = jax.lax.dot_general(q, k, (((1,), (1,)), ((), ())),
                              preferred_element_type=jnp.float32) * scale
    rows = i * br + jax.lax.broadcasted_iota(jnp.int32, (br, n), 0)
    cols = jax.lax.broadcasted_iota(jnp.int32, (br, n), 1)
    in_band = jnp.abs(rows - cols) <= window
    slab = br + 2 * window if br + 2 * window <= n else n
    start = jnp.clip(i * br - (slab - br) // 2, 0, n - slab)
    ks = k_ref[0, pl.ds(start, slab), :]
    vs = v_ref[0, pl.ds(start, slab), :]
    sl = jax.lax.dot_general(q, ks, (((1,), (1,)), ((), ())),
                             preferred_element_type=jnp.float32) * scale
    srows = i * br + jax.lax.broadcasted_iota(jnp.int32, (br, slab), 0)
    scols = start + jax.lax.broadcasted_iota(jnp.int32, (br, slab), 1)
    sband = jnp.abs(srows - scols) <= window
    a = jnp.where(sband, sl, -jnp.inf)
    m = jnp.max(a, axis=-1, keepdims=True)
    p = jnp.exp(a - m)
    p = p * (1.0 / jnp.sum(p, axis=-1, keepdims=True))
    ylocal_ref[0] = jnp.dot(p, vs, preferred_element_type=jnp.float32)
    x = jnp.where(in_band, -jnp.inf, sim)
    t32 = _nth_largest(x, topk)
    w = jnp.where((x >= t32) & (x > 0.0), x, 0.0)
    denom = jnp.sum(w, axis=-1, keepdims=True) + 1e-9
    s_ref[0] = ((1.0 - alpha) * w / denom).astype(jnp.bfloat16)


def _mm1_kernel(sl_ref, sr_ref, a_ref, b_ref, acc_ref, srow_ref, *, nk):
    i = pl.program_id(1)
    kk = pl.program_id(2)
    @pl.when(kk == 0)
    def _():
        acc_ref[...] = jnp.zeros_like(acc_ref)
    acc_ref[...] += jnp.dot(sl_ref[0], sr_ref[0],
                            preferred_element_type=jnp.float32)
    @pl.when(kk == i)
    def _():
        srow_ref[...] = sr_ref[0]
    @pl.when(kk == nk - 1)
    def _():
        acc = acc_ref[...]
        a_ref[0] = acc.astype(jnp.bfloat16)
        b_ref[0] = (acc + srow_ref[...].astype(jnp.float32)).astype(
            jnp.bfloat16)


def _mm2_final_kernel(a_ref, b_ref, v_ref, ylocal_ref, o_ref, acc_ref,
                      brow_ref, *, nk, br, topk, coef):
    i = pl.program_id(1)
    kk = pl.program_id(2)
    @pl.when(kk == 0)
    def _():
        acc_ref[...] = jnp.zeros_like(acc_ref)
    acc_ref[...] += jnp.dot(a_ref[0], b_ref[0],
                            preferred_element_type=jnp.float32)
    @pl.when(kk == i)
    def _():
        brow_ref[...] = b_ref[0]
    @pl.when(kk == nk - 1)
    def _():
        n = acc_ref.shape[-1]
        rows = i * br + jax.lax.broadcasted_iota(jnp.int32, (br, n), 0)
        cols = jax.lax.broadcasted_iota(jnp.int32, (br, n), 1)
        u = (acc_ref[...] + brow_ref[...].astype(jnp.float32)
             + (rows == cols).astype(jnp.float32))
        t32 = _nth_largest(u, topk)
        w = jnp.where(u >= t32, u, 0.0)
        yr = jnp.dot(w, v_ref[0], preferred_element_type=jnp.float32)
        o_ref[0] = ylocal_ref[0] + coef * yr


def kernel(q, k, v):
    B, H, N, D = q.shape
    Z = B * H
    br = min(_BR, N)
    brm = min(_BRM, N)
    bk = brm
    qf = q.reshape(Z, N, D)
    kf = k.reshape(Z, N, D)
    vf = v.reshape(Z, N, D)

    ylocal, S = pl.pallas_call(
        functools.partial(_stage1_kernel, topk=_TOPK, window=_WINDOW,
                          alpha=_ALPHA, br=br),
        grid=(Z, N // br),
        in_specs=[
            pl.BlockSpec((1, br, D), lambda z, i: (z, i, 0)),
            pl.BlockSpec((1, N, D), lambda z, i: (z, 0, 0)),
            pl.BlockSpec((1, N, D), lambda z, i: (z, 0, 0)),
        ],
        out_specs=[
            pl.BlockSpec((1, br, D), lambda z, i: (z, i, 0)),
            pl.BlockSpec((1, br, N), lambda z, i: (z, i, 0)),
        ],
        out_shape=[
            jax.ShapeDtypeStruct((Z, N, D), jnp.float32),
            jax.ShapeDtypeStruct((Z, N, N), jnp.bfloat16),
        ],
        compiler_params=pltpu.CompilerParams(
            dimension_semantics=("parallel", "parallel")),
    )(qf, kf, vf)

    nk = N // bk
    A, Bm = pl.pallas_call(
        functools.partial(_mm1_kernel, nk=nk),
        grid=(Z, N // brm, nk),
        in_specs=[
            pl.BlockSpec((1, brm, bk), lambda z, i, kk: (z, i, kk)),
            pl.BlockSpec((1, bk, N), lambda z, i, kk: (z, kk, 0)),
        ],
        out_specs=[
            pl.BlockSpec((1, brm, N), lambda z, i, kk: (z, i, 0)),
            pl.BlockSpec((1, brm, N), lambda z, i, kk: (z, i, 0)),
        ],
        out_shape=[
            jax.ShapeDtypeStruct((Z, N, N), jnp.bfloat16),
            jax.ShapeDtypeStruct((Z, N, N), jnp.bfloat16),
        ],
        scratch_shapes=[pltpu.VMEM((brm, N), jnp.float32),
                        pltpu.VMEM((bk, N), jnp.bfloat16)],
        compiler_params=pltpu.CompilerParams(
            dimension_semantics=("parallel", "parallel", "arbitrary")),
    )(S, S)

    out = pl.pallas_call(
        functools.partial(_mm2_final_kernel, nk=nk, br=brm, topk=_TOPK,
                          coef=_LENS * _ALPHA),
        grid=(Z, N // brm, nk),
        in_specs=[
            pl.BlockSpec((1, brm, bk), lambda z, i, kk: (z, i, kk)),
            pl.BlockSpec((1, bk, N), lambda z, i, kk: (z, kk, 0)),
            pl.BlockSpec((1, N, D), lambda z, i, kk: (z, 0, 0)),
            pl.BlockSpec((1, brm, D), lambda z, i, kk: (z, i, 0)),
        ],
        out_specs=pl.BlockSpec((1, brm, D), lambda z, i, kk: (z, i, 0)),
        out_shape=jax.ShapeDtypeStruct((Z, N, D), jnp.float32),
        scratch_shapes=[pltpu.VMEM((brm, N), jnp.float32),
                        pltpu.VMEM((bk, N), jnp.bfloat16)],
        compiler_params=pltpu.CompilerParams(
            dimension_semantics=("parallel", "parallel", "arbitrary")),
    )(A, Bm, vf, ylocal)

    return out.reshape(B, H, N, D)
